# baseline (speedup 1.0000x reference)
import functools

import jax
import jax.numpy as jnp
from jax.experimental import pallas as pl
from jax.experimental.pallas import tpu as pltpu

_N_MAX_PIXELS = 20000
_THRESHOLD = 0.5


def _count_body(x_ref, carr_ref, acc_ref, *, hb, w):
    bi = pl.program_id(0)
    i = pl.program_id(1)

    @pl.when(i == 0)
    def _():
        acc_ref[0] = 0

    f = (x_ref[0][:, 0, :] > _THRESHOLD).astype(jnp.float32)
    s = jnp.sum(f).astype(jnp.int32)
    carr_ref[bi, i] = acc_ref[0]
    acc_ref[0] = acc_ref[0] + s


def _main_body(carr_ref, fetch_ref, x_ref, out_ref, m_ref, *, hb, w, c):
    bi = pl.program_id(0)
    i = pl.program_id(1)
    carry = carr_ref[bi, i]
    skip = carry >= _N_MAX_PIXELS

    @pl.when(jnp.logical_not(skip))
    def _():
        x = x_ref[0]
        f = (x[:, 0, :] > _THRESHOLD).astype(jnp.float32)
        s = jnp.sum(f).astype(jnp.int32)

        keep_all = (carry + s <= _N_MAX_PIXELS).astype(jnp.float32)
        m = f * keep_all
        m_ref[0] = m
        out_ref[0] = x * m[:, None, :]

        @pl.when(jnp.logical_and(carry + s > _N_MAX_PIXELS, carry < _N_MAX_PIXELS))
        def _():
            a = f
            d = 1
            while d < w:
                a = a + jnp.concatenate(
                    [jnp.zeros((hb, d), jnp.float32), a[:, : w - d]], axis=1
                )
                d *= 2
            rs = a[:, w - 1 : w]
            rincl = rs
            d = 1
            while d < hb:
                rincl = rincl + jnp.concatenate(
                    [jnp.zeros((d, 1), jnp.float32), rincl[: hb - d]], axis=0
                )
                d *= 2
            total = a + (rincl - rs)
            limit = (_N_MAX_PIXELS - carry) + 0.5
            mb = f * (total < limit).astype(jnp.float32)
            m_ref[0] = mb
            out_ref[0] = x * mb[:, None, :]

    @pl.when(skip)
    def _():
        m_ref[0] = jnp.zeros((hb, w), jnp.float32)
        out_ref[0] = jnp.zeros((hb, c, w), jnp.float32)


def kernel(inputs):
    b, h, w, c = inputs.shape
    hb = 32
    nblk = h // hb
    xt = inputs.transpose(0, 1, 3, 2)
    grid = (b, nblk)

    carries = pl.pallas_call(
        functools.partial(_count_body, hb=hb, w=w),
        grid=grid,
        in_specs=[pl.BlockSpec((1, hb, 8, w), lambda bi, i: (bi, i, 0, 0))],
        out_specs=pl.BlockSpec(memory_space=pltpu.MemorySpace.SMEM),
        out_shape=jax.ShapeDtypeStruct((b, nblk), jnp.int32),
        scratch_shapes=[pltpu.SMEM((1,), jnp.int32)],
        compiler_params=pltpu.CompilerParams(
            dimension_semantics=("arbitrary", "arbitrary")
        ),
    )(xt)

    lastneeded = jnp.maximum(
        jnp.sum((carries < _N_MAX_PIXELS).astype(jnp.int32), axis=1) - 1, 0
    )
    fetchidx = jnp.minimum(
        jnp.arange(nblk, dtype=jnp.int32)[None, :], lastneeded[:, None]
    )

    grid_spec = pltpu.PrefetchScalarGridSpec(
        num_scalar_prefetch=2,
        grid=grid,
        in_specs=[
            pl.BlockSpec((1, hb, c, w), lambda bi, i, carr, fetch: (bi, fetch[bi, i], 0, 0)),
        ],
        out_specs=[
            pl.BlockSpec((1, hb, c, w), lambda bi, i, carr, fetch: (bi, i, 0, 0)),
            pl.BlockSpec((1, hb, w), lambda bi, i, carr, fetch: (bi, i, 0)),
        ],
        scratch_shapes=[],
    )

    out_t, mask = pl.pallas_call(
        functools.partial(_main_body, hb=hb, w=w, c=c),
        grid_spec=grid_spec,
        out_shape=[
            jax.ShapeDtypeStruct((b, h, c, w), inputs.dtype),
            jax.ShapeDtypeStruct((b, h, w), inputs.dtype),
        ],
        compiler_params=pltpu.CompilerParams(
            dimension_semantics=("arbitrary", "arbitrary")
        ),
    )(carries, fetchidx, xt)

    return out_t.transpose(0, 1, 3, 2), mask.reshape(b, h, w, 1)
